# baseline (speedup 1.0000x reference)
import functools

import jax
import jax.numpy as jnp
from jax import lax
from jax.experimental import pallas as pl
from jax.experimental.pallas import tpu as pltpu
from jax.experimental.pallas import tpu_sc as plsc

N = 10000
E = 320000
IN_CH = 128
OUT_CH = 128
HID = 2 * OUT_CH

NC = 2
NS = 16
B = 128

NPAD = 10240
TRASH = N
STRIPE = NPAD // NS

CH = 16
NBLK1 = 160
NBLK2 = 80

_mesh = plsc.VectorSubcoreMesh(
    core_axis_name="c", subcore_axis_name="s", num_cores=NC, num_subcores=NS)


def _pad_edges(idx, n_chunks, nblk):
    total = n_chunks * nblk * B
    idx = jnp.concatenate(
        [idx, jnp.full((total - E,), TRASH, dtype=jnp.int32)])
    return idx.reshape(n_chunks, nblk, B)


def _deg_body(col_hbm, zeros_hbm, ones_hbm, degp_hbm, col_v, ones_v, deg_sp,
              sem):
    c = lax.axis_index("c")
    s = lax.axis_index("s")
    base = s * STRIPE
    pltpu.sync_copy(zeros_hbm.at[pl.ds(base, STRIPE)],
                    deg_sp.at[pl.ds(base, STRIPE)])
    pltpu.sync_copy(ones_hbm, ones_v)
    plsc.subcore_barrier()

    def chunk(k, _):
        pltpu.sync_copy(col_hbm.at[c, s, pl.ds(k * CH, CH)], col_v)
        descs = [
            pltpu.async_copy(ones_v, deg_sp.at[col_v.at[j]], sem, add=True)
            for j in range(CH)
        ]
        for d in descs:
            d.wait()
        return 0

    lax.fori_loop(0, NBLK2 // CH, chunk, 0)
    plsc.subcore_barrier()
    pltpu.sync_copy(deg_sp.at[pl.ds(base, STRIPE)],
                    degp_hbm.at[c, pl.ds(base, STRIPE)])


_deg_kernel = functools.partial(
    pl.kernel,
    out_type=jax.ShapeDtypeStruct((NC, NPAD, 128), jnp.float32),
    mesh=_mesh,
    scratch_types=[
        pltpu.VMEM((CH, B), jnp.int32),
        pltpu.VMEM((B, 128), jnp.float32),
        pltpu.VMEM_SHARED((NPAD, 128), jnp.float32),
        pltpu.SemaphoreType.DMA,
    ],
)(_deg_body)


def _agg_body(nblk, init_mul, table_hbm, row_hbm, col_hbm, out_hbm,
              row_v, col_v, buf0, buf1, out_sp, sem0, sem1):
    c = lax.axis_index("c")
    s = lax.axis_index("s")
    base = s * STRIPE
    pltpu.sync_copy(table_hbm.at[pl.ds(init_mul * c * NPAD + base, STRIPE)],
                    out_sp.at[pl.ds(base, STRIPE)])
    plsc.subcore_barrier()
    bufs = (buf0, buf1)
    sems = (sem0, sem1)

    def chunk(k, _):
        pltpu.sync_copy(row_hbm.at[c, s, pl.ds(k * CH, CH)], row_v)
        pltpu.sync_copy(col_hbm.at[c, s, pl.ds(k * CH, CH)], col_v)
        gat = pltpu.async_copy(table_hbm.at[row_v.at[0]], bufs[0], sems[0])
        for j in range(CH):
            nxt = None
            if j + 1 < CH:
                nxt = pltpu.async_copy(
                    table_hbm.at[row_v.at[j + 1]], bufs[(j + 1) % 2],
                    sems[(j + 1) % 2])
            gat.wait()
            pltpu.sync_copy(bufs[j % 2], out_sp.at[col_v.at[j]], add=True)
            gat = nxt
        return 0

    lax.fori_loop(0, nblk // CH, chunk, 0)
    plsc.subcore_barrier()
    pltpu.sync_copy(out_sp.at[pl.ds(base, STRIPE)],
                    out_hbm.at[pl.ds(c * NPAD + base, STRIPE)])


def _make_agg(nblk, init_mul):
    return functools.partial(
        pl.kernel,
        out_type=jax.ShapeDtypeStruct((2 * NPAD, 128), jnp.float32),
        mesh=_mesh,
        scratch_types=[
            pltpu.VMEM((CH, B), jnp.int32),
            pltpu.VMEM((CH, B), jnp.int32),
            pltpu.VMEM((B, 128), jnp.float32),
            pltpu.VMEM((B, 128), jnp.float32),
            pltpu.VMEM_SHARED((NPAD, 128), jnp.float32),
            pltpu.SemaphoreType.DMA,
            pltpu.SemaphoreType.DMA,
        ],
    )(functools.partial(_agg_body, nblk, init_mul))


_agg1 = _make_agg(NBLK1, 1)
_agg2 = _make_agg(NBLK2, 1)


BN = 1024
NB = NPAD // BN


def _dinv_of(degp):
    deg = degp[0, :, 0] + degp[1, :, 0] + 1.0
    return lax.rsqrt(deg)


def _k1a_body(x_ref, w1_ref, out_ref):
    out_ref[0] = jnp.dot(x_ref[...], w1_ref[...],
                         preferred_element_type=jnp.float32)


def _tc1a(x_pad, W1):
    return pl.pallas_call(
        _k1a_body,
        grid=(NB, 2),
        in_specs=[
            pl.BlockSpec((BN, IN_CH), lambda i, c: (i, 0)),
            pl.BlockSpec((IN_CH, 128), lambda i, c: (0, c)),
        ],
        out_specs=pl.BlockSpec((1, BN, 128), lambda i, c: (c, i, 0)),
        out_shape=jax.ShapeDtypeStruct((2, NPAD, 128), jnp.float32),
    )(x_pad, W1)


def _k1b_body(xw_ref, degp_ref, out_ref):
    dinv = _dinv_of(degp_ref[...])[None, :, None]
    out_ref[...] = dinv * xw_ref[...]


def _tc1b(xw1, degp):
    return pl.pallas_call(
        _k1b_body,
        grid=(NB,),
        in_specs=[
            pl.BlockSpec((2, BN, 128), lambda i: (0, i, 0)),
            pl.BlockSpec((NC, BN, 128), lambda i: (0, i, 0)),
        ],
        out_specs=pl.BlockSpec((2, BN, 128), lambda i: (0, i, 0)),
        out_shape=jax.ShapeDtypeStruct((2, NPAD, 128), jnp.float32),
    )(xw1, degp)


def _k2_body(s1_ref, degp_ref, b1_ref, w2_ref, out_ref):
    dinv = _dinv_of(degp_ref[...])[:, None]
    h0 = jax.nn.relu(dinv * s1_ref[0] + b1_ref[0][None, :])
    h1 = jax.nn.relu(dinv * s1_ref[1] + b1_ref[1][None, :])
    xw = (jnp.dot(h0, w2_ref[0], preferred_element_type=jnp.float32)
          + jnp.dot(h1, w2_ref[1], preferred_element_type=jnp.float32))
    xs2 = dinv * xw
    out_ref[0] = xs2
    out_ref[1] = xs2


def _tc2(S1, degp, b1r, W2r):
    return pl.pallas_call(
        _k2_body,
        grid=(NB,),
        in_specs=[
            pl.BlockSpec((2, BN, 128), lambda i: (0, i, 0)),
            pl.BlockSpec((NC, BN, 128), lambda i: (0, i, 0)),
            pl.BlockSpec((2, 128), lambda i: (0, 0)),
            pl.BlockSpec((2, 128, 128), lambda i: (0, 0, 0)),
        ],
        out_specs=pl.BlockSpec((2, BN, 128), lambda i: (0, i, 0)),
        out_shape=jax.ShapeDtypeStruct((2, NPAD, 128), jnp.float32),
    )(S1, degp, b1r, W2r)


def _k3_body(p_ref, xs2_ref, degp_ref, b2_ref, out_ref):
    dinv = _dinv_of(degp_ref[...])[:, None]
    agg = p_ref[0] + p_ref[1] - xs2_ref[...]
    out_ref[...] = dinv * agg + b2_ref[0][None, :]


def _tc3(P, xs2, degp, b2r):
    return pl.pallas_call(
        _k3_body,
        grid=(NB,),
        in_specs=[
            pl.BlockSpec((2, BN, 128), lambda i: (0, i, 0)),
            pl.BlockSpec((BN, 128), lambda i: (i, 0)),
            pl.BlockSpec((NC, BN, 128), lambda i: (0, i, 0)),
            pl.BlockSpec((1, 128), lambda i: (0, 0)),
        ],
        out_specs=pl.BlockSpec((BN, 128), lambda i: (i, 0)),
        out_shape=jax.ShapeDtypeStruct((NPAD, 128), jnp.float32),
    )(P, xs2, degp, b2r)


def kernel(x, edge_index, W1, b1, W2, b2):
    row = edge_index[0].astype(jnp.int32)
    col = edge_index[1].astype(jnp.int32)

    rowA = _pad_edges(row, NC * NS, NBLK2).reshape(NC, NS, NBLK2, B)
    colA = _pad_edges(col, NC * NS, NBLK2).reshape(NC, NS, NBLK2, B)
    row1_base = _pad_edges(row, NS, NBLK1)
    row1 = jnp.stack([row1_base, row1_base + NPAD])
    col1_base = _pad_edges(col, NS, NBLK1)
    col1 = jnp.stack([col1_base, col1_base])

    x_pad = jnp.pad(x, ((0, NPAD - N), (0, 0)))
    b1r = b1.reshape(2, 128)
    b2r = b2.reshape(1, 128)
    W2r = W2.reshape(2, 128, 128)

    degp = _deg_kernel(colA, jnp.zeros((NPAD, 128), jnp.float32),
                   jnp.ones((B, 128), jnp.float32))
    xw1 = _tc1a(x_pad, W1)
    xs1 = _tc1b(xw1, degp)
    S1 = _agg1(xs1.reshape(2 * NPAD, 128), row1, col1)
    xs2 = _tc2(S1.reshape(2, NPAD, 128), degp, b1r, W2r)
    rowA2 = rowA + jnp.arange(NC, dtype=jnp.int32).reshape(NC, 1, 1, 1) * NPAD
    P = _agg2(xs2.reshape(2 * NPAD, 128), rowA2, colA)
    out = _tc3(P.reshape(2, NPAD, 128), xs2[0], degp, b2r)
    return out[:N]


# speedup vs baseline: 1.1747x; 1.1747x over previous
import functools

import jax
import jax.numpy as jnp
from jax import lax
from jax.experimental import pallas as pl
from jax.experimental.pallas import tpu as pltpu
from jax.experimental.pallas import tpu_sc as plsc

N = 10000
E = 320000
IN_CH = 128
OUT_CH = 128
HID = 2 * OUT_CH

NC = 2
NS = 16
B = 128

NPAD = 10240
TRASH = N
STRIPE = NPAD // NS

CH = 16
NBLK1 = 160
NBLK2 = 80
DW = 128

_mesh = plsc.VectorSubcoreMesh(
    core_axis_name="c", subcore_axis_name="s", num_cores=NC, num_subcores=NS)


def _pad_edges(idx, n_chunks, nblk):
    total = n_chunks * nblk * B
    idx = jnp.concatenate(
        [idx, jnp.full((total - E,), TRASH, dtype=jnp.int32)])
    return idx.reshape(n_chunks, nblk, B)


def _deg_body(col_hbm, zeros_hbm, ones_hbm, degp_hbm, col_v, ones_v, deg_sp,
              sem):
    c = lax.axis_index("c")
    s = lax.axis_index("s")
    base = s * STRIPE
    pltpu.sync_copy(zeros_hbm.at[pl.ds(base, STRIPE)],
                    deg_sp.at[pl.ds(base, STRIPE)])
    pltpu.sync_copy(ones_hbm, ones_v)
    plsc.subcore_barrier()

    def chunk(k, _):
        pltpu.sync_copy(col_hbm.at[c, s, pl.ds(k * CH, CH)], col_v)
        descs = [
            pltpu.async_copy(ones_v, deg_sp.at[col_v.at[j]], sem, add=True)
            for j in range(CH)
        ]
        for d in descs:
            d.wait()
        return 0

    lax.fori_loop(0, NBLK2 // CH, chunk, 0)
    plsc.subcore_barrier()
    pltpu.sync_copy(deg_sp.at[pl.ds(base, STRIPE)],
                    degp_hbm.at[c, pl.ds(base, STRIPE)])


_deg_kernel = functools.partial(
    pl.kernel,
    out_type=jax.ShapeDtypeStruct((NC, NPAD, DW), jnp.float32),
    mesh=_mesh,
    scratch_types=[
        pltpu.VMEM((CH, B), jnp.int32),
        pltpu.VMEM((B, DW), jnp.float32),
        pltpu.VMEM_SHARED((NPAD, DW), jnp.float32),
        pltpu.SemaphoreType.DMA,
    ],
)(_deg_body)


def _agg_body(nblk, init_mul, table_hbm, row_hbm, col_hbm, out_hbm,
              row_v, col_v, buf0, buf1, out_sp, sem0, sem1, isem):
    c = lax.axis_index("c")
    s = lax.axis_index("s")
    base = s * STRIPE
    pltpu.sync_copy(table_hbm.at[pl.ds(init_mul * c * NPAD + base, STRIPE)],
                    out_sp.at[pl.ds(base, STRIPE)])
    pltpu.sync_copy(row_hbm.at[c, s, pl.ds(0, CH)], row_v.at[0])
    pltpu.sync_copy(col_hbm.at[c, s, pl.ds(0, CH)], col_v.at[0])
    plsc.subcore_barrier()
    bufs = (buf0, buf1)
    sems = (sem0, sem1)
    nchk = nblk // CH
    stage = []
    gat = pltpu.async_copy(table_hbm.at[row_v.at[0, 0]], bufs[0], sems[0])
    for j in range(nblk):
        ch, pos = divmod(j, CH)
        if pos == 0 and ch + 1 < nchk:
            nxt_p = (ch + 1) % 2
            stage = [
                pltpu.async_copy(row_hbm.at[c, s, pl.ds((ch + 1) * CH, CH)],
                                 row_v.at[nxt_p], isem),
                pltpu.async_copy(col_hbm.at[c, s, pl.ds((ch + 1) * CH, CH)],
                                 col_v.at[nxt_p], isem),
            ]
        nxt = None
        if j + 1 < nblk:
            ch1, pos1 = divmod(j + 1, CH)
            if pos1 == 0:
                for d in stage:
                    d.wait()
                stage = []
            nxt = pltpu.async_copy(
                table_hbm.at[row_v.at[ch1 % 2, pos1]], bufs[(j + 1) % 2],
                sems[(j + 1) % 2])
        gat.wait()
        pltpu.sync_copy(bufs[j % 2], out_sp.at[col_v.at[ch % 2, pos]],
                        add=True)
        gat = nxt
    plsc.subcore_barrier()
    pltpu.sync_copy(out_sp.at[pl.ds(base, STRIPE)],
                    out_hbm.at[pl.ds(c * NPAD + base, STRIPE)])


def _make_agg(nblk, init_mul):
    return functools.partial(
        pl.kernel,
        out_type=jax.ShapeDtypeStruct((2 * NPAD, 128), jnp.float32),
        mesh=_mesh,
        scratch_types=[
            pltpu.VMEM((2, CH, B), jnp.int32),
            pltpu.VMEM((2, CH, B), jnp.int32),
            pltpu.VMEM((B, 128), jnp.float32),
            pltpu.VMEM((B, 128), jnp.float32),
            pltpu.VMEM_SHARED((NPAD, 128), jnp.float32),
            pltpu.SemaphoreType.DMA,
            pltpu.SemaphoreType.DMA,
            pltpu.SemaphoreType.DMA,
        ],
    )(functools.partial(_agg_body, nblk, init_mul))


_agg1 = _make_agg(NBLK1, 1)
_agg2 = _make_agg(NBLK2, 0)


BN = 1024
NB = NPAD // BN


def _dinv_of(degp):
    deg = degp[0, :, 0] + degp[1, :, 0] + 1.0
    return lax.rsqrt(deg)


def _k1a_body(x_ref, w1_ref, out_ref):
    out_ref[0] = jnp.dot(x_ref[...], w1_ref[...],
                         preferred_element_type=jnp.float32)


def _tc1a(x_pad, W1):
    return pl.pallas_call(
        _k1a_body,
        grid=(NB, 2),
        in_specs=[
            pl.BlockSpec((BN, IN_CH), lambda i, c: (i, 0)),
            pl.BlockSpec((IN_CH, 128), lambda i, c: (0, c)),
        ],
        out_specs=pl.BlockSpec((1, BN, 128), lambda i, c: (c, i, 0)),
        out_shape=jax.ShapeDtypeStruct((2, NPAD, 128), jnp.float32),
    )(x_pad, W1)


def _k1b_body(xw_ref, degp_ref, out_ref):
    dinv = _dinv_of(degp_ref[...])[None, :, None]
    out_ref[...] = dinv * xw_ref[...]


def _tc1b(xw1, degp):
    return pl.pallas_call(
        _k1b_body,
        grid=(NB,),
        in_specs=[
            pl.BlockSpec((2, BN, 128), lambda i: (0, i, 0)),
            pl.BlockSpec((NC, BN, DW), lambda i: (0, i, 0)),
        ],
        out_specs=pl.BlockSpec((2, BN, 128), lambda i: (0, i, 0)),
        out_shape=jax.ShapeDtypeStruct((2, NPAD, 128), jnp.float32),
    )(xw1, degp)


def _k2_body(s1_ref, degp_ref, b1_ref, w2_ref, out_ref):
    dinv = _dinv_of(degp_ref[...])[:, None]
    h0 = jax.nn.relu(dinv * s1_ref[0] + b1_ref[0][None, :])
    h1 = jax.nn.relu(dinv * s1_ref[1] + b1_ref[1][None, :])
    xw = (jnp.dot(h0, w2_ref[0], preferred_element_type=jnp.float32)
          + jnp.dot(h1, w2_ref[1], preferred_element_type=jnp.float32))
    out_ref[...] = dinv * xw


def _tc2(S1, degp, b1r, W2r):
    return pl.pallas_call(
        _k2_body,
        grid=(NB,),
        in_specs=[
            pl.BlockSpec((2, BN, 128), lambda i: (0, i, 0)),
            pl.BlockSpec((NC, BN, DW), lambda i: (0, i, 0)),
            pl.BlockSpec((2, 128), lambda i: (0, 0)),
            pl.BlockSpec((2, 128, 128), lambda i: (0, 0, 0)),
        ],
        out_specs=pl.BlockSpec((BN, 128), lambda i: (i, 0)),
        out_shape=jax.ShapeDtypeStruct((NPAD, 128), jnp.float32),
    )(S1, degp, b1r, W2r)


def _k3_body(p_ref, xs2_ref, degp_ref, b2_ref, out_ref):
    dinv = _dinv_of(degp_ref[...])[:, None]
    agg = p_ref[0] + p_ref[1] - xs2_ref[...]
    out_ref[...] = dinv * agg + b2_ref[0][None, :]


def _tc3(P, xs2, degp, b2r):
    return pl.pallas_call(
        _k3_body,
        grid=(NB,),
        in_specs=[
            pl.BlockSpec((2, BN, 128), lambda i: (0, i, 0)),
            pl.BlockSpec((BN, 128), lambda i: (i, 0)),
            pl.BlockSpec((NC, BN, DW), lambda i: (0, i, 0)),
            pl.BlockSpec((1, 128), lambda i: (0, 0)),
        ],
        out_specs=pl.BlockSpec((BN, 128), lambda i: (i, 0)),
        out_shape=jax.ShapeDtypeStruct((NPAD, 128), jnp.float32),
    )(P, xs2, degp, b2r)


def kernel(x, edge_index, W1, b1, W2, b2):
    row = edge_index[0].astype(jnp.int32)
    col = edge_index[1].astype(jnp.int32)

    rowA = _pad_edges(row, NC * NS, NBLK2).reshape(NC, NS, NBLK2, B)
    colA = _pad_edges(col, NC * NS, NBLK2).reshape(NC, NS, NBLK2, B)
    row1_base = _pad_edges(row, NS, NBLK1)
    row1 = jnp.stack([row1_base, row1_base + NPAD])
    col1_base = _pad_edges(col, NS, NBLK1)
    col1 = jnp.stack([col1_base, col1_base])

    x_pad = jnp.pad(x, ((0, NPAD - N), (0, 0)))
    b1r = b1.reshape(2, 128)
    b2r = b2.reshape(1, 128)
    W2r = W2.reshape(2, 128, 128)

    degp = _deg_kernel(colA, jnp.zeros((NPAD, DW), jnp.float32),
                   jnp.ones((B, DW), jnp.float32))
    xw1 = _tc1a(x_pad, W1)
    xs1 = _tc1b(xw1, degp)
    S1 = _agg1(xs1.reshape(2 * NPAD, 128), row1, col1)
    xs2 = _tc2(S1.reshape(2, NPAD, 128), degp, b1r, W2r)
    P = _agg2(xs2, rowA, colA)
    out = _tc3(P.reshape(2, NPAD, 128), xs2, degp, b2r)
    return out[:N]


# speedup vs baseline: 2.0560x; 1.7502x over previous
import functools

import jax
import jax.numpy as jnp
from jax import lax
from jax.experimental import pallas as pl
from jax.experimental.pallas import tpu as pltpu
from jax.experimental.pallas import tpu_sc as plsc

N = 10000
E = 320000
IN_CH = 128
OUT_CH = 128
HID = 2 * OUT_CH

NC = 2
NS = 16
B = 128

NPAD = 10240
TRASH = N
STRIPE = NPAD // NS

CH = 16
NBLK1 = 160
NPROC1 = 157
NBLK2 = 80
NPROC2 = 79
DW = 128

_mesh = plsc.VectorSubcoreMesh(
    core_axis_name="c", subcore_axis_name="s", num_cores=NC, num_subcores=NS)


def _pad_edges(idx, n_chunks, nblk):
    per = E // n_chunks
    idx = idx.reshape(n_chunks, per)
    idx = jnp.pad(idx, ((0, 0), (0, nblk * B - per)), constant_values=TRASH)
    return idx.reshape(n_chunks, nblk, B)


def _deg_body(col_hbm, zeros_hbm, ones_hbm, degp_hbm, col_v, ones_v, deg_sp,
              sem):
    c = lax.axis_index("c")
    s = lax.axis_index("s")
    base = s * STRIPE
    pltpu.sync_copy(zeros_hbm.at[pl.ds(base, STRIPE)],
                    deg_sp.at[pl.ds(base, STRIPE)])
    pltpu.sync_copy(ones_hbm, ones_v)
    plsc.subcore_barrier()

    nchk = -(-NPROC2 // CH)
    for k in range(nchk):
        pltpu.sync_copy(col_hbm.at[c, s, pl.ds(k * CH, CH)], col_v)
        nb = min(CH, NPROC2 - k * CH)
        descs = [
            pltpu.async_copy(ones_v, deg_sp.at[col_v.at[j]], sem, add=True)
            for j in range(nb)
        ]
        for d in descs:
            d.wait()
    plsc.subcore_barrier()
    pltpu.sync_copy(deg_sp.at[pl.ds(base, STRIPE)],
                    degp_hbm.at[c, pl.ds(base, STRIPE)])


_deg_kernel = functools.partial(
    pl.kernel,
    out_type=jax.ShapeDtypeStruct((NC, NPAD, DW), jnp.float32),
    mesh=_mesh,
    scratch_types=[
        pltpu.VMEM((CH, B), jnp.int32),
        pltpu.VMEM((B, DW), jnp.float32),
        pltpu.VMEM_SHARED((NPAD, DW), jnp.float32),
        pltpu.SemaphoreType.DMA,
    ],
)(_deg_body)


def _agg_body(nblk, init_mul, table_hbm, row_hbm, col_hbm, out_hbm,
              row_v, col_v, buf0, buf1, out_sp, sem0, sem1, isem):
    c = lax.axis_index("c")
    s = lax.axis_index("s")
    base = s * STRIPE
    pltpu.sync_copy(table_hbm.at[pl.ds(init_mul * c * NPAD + base, STRIPE)],
                    out_sp.at[pl.ds(base, STRIPE)])
    pltpu.sync_copy(row_hbm.at[c, s, pl.ds(0, CH)], row_v.at[0])
    pltpu.sync_copy(col_hbm.at[c, s, pl.ds(0, CH)], col_v.at[0])
    plsc.subcore_barrier()
    bufs = (buf0, buf1)
    sems = (sem0, sem1)
    nchk = -(-nblk // CH)
    stage = []
    gat = pltpu.async_copy(table_hbm.at[row_v.at[0, 0]], bufs[0], sems[0])
    for j in range(nblk):
        ch, pos = divmod(j, CH)
        if pos == 0 and ch + 1 < nchk:
            nxt_p = (ch + 1) % 2
            stage = [
                pltpu.async_copy(row_hbm.at[c, s, pl.ds((ch + 1) * CH, CH)],
                                 row_v.at[nxt_p], isem),
                pltpu.async_copy(col_hbm.at[c, s, pl.ds((ch + 1) * CH, CH)],
                                 col_v.at[nxt_p], isem),
            ]
        nxt = None
        if j + 1 < nblk:
            ch1, pos1 = divmod(j + 1, CH)
            if pos1 == 0:
                for d in stage:
                    d.wait()
                stage = []
            nxt = pltpu.async_copy(
                table_hbm.at[row_v.at[ch1 % 2, pos1]], bufs[(j + 1) % 2],
                sems[(j + 1) % 2])
        gat.wait()
        pltpu.sync_copy(bufs[j % 2], out_sp.at[col_v.at[ch % 2, pos]],
                        add=True)
        gat = nxt
    plsc.subcore_barrier()
    pltpu.sync_copy(out_sp.at[pl.ds(base, STRIPE)],
                    out_hbm.at[pl.ds(c * NPAD + base, STRIPE)])


def _make_agg(nblk, init_mul):
    return functools.partial(
        pl.kernel,
        out_type=jax.ShapeDtypeStruct((2 * NPAD, 128), jnp.float32),
        mesh=_mesh,
        scratch_types=[
            pltpu.VMEM((2, CH, B), jnp.int32),
            pltpu.VMEM((2, CH, B), jnp.int32),
            pltpu.VMEM((B, 128), jnp.float32),
            pltpu.VMEM((B, 128), jnp.float32),
            pltpu.VMEM_SHARED((NPAD, 128), jnp.float32),
            pltpu.SemaphoreType.DMA,
            pltpu.SemaphoreType.DMA,
            pltpu.SemaphoreType.DMA,
        ],
    )(functools.partial(_agg_body, nblk, init_mul))


_agg1 = _make_agg(NPROC1, 1)
_agg2 = _make_agg(NPROC2, 0)


BN = 1024
NB = NPAD // BN


def _dinv_of(degp):
    deg = degp[0, :, 0] + degp[1, :, 0] + 1.0
    return lax.rsqrt(deg)


def _k1a_body(x_ref, w1_ref, out_ref):
    out_ref[0] = jnp.dot(x_ref[...], w1_ref[...],
                         preferred_element_type=jnp.float32)


def _tc1a(x_pad, W1):
    return pl.pallas_call(
        _k1a_body,
        grid=(NB, 2),
        in_specs=[
            pl.BlockSpec((BN, IN_CH), lambda i, c: (i, 0)),
            pl.BlockSpec((IN_CH, 128), lambda i, c: (0, c)),
        ],
        out_specs=pl.BlockSpec((1, BN, 128), lambda i, c: (c, i, 0)),
        out_shape=jax.ShapeDtypeStruct((2, NPAD, 128), jnp.float32),
    )(x_pad, W1)


def _k1b_body(xw_ref, degp_ref, out_ref):
    dinv = _dinv_of(degp_ref[...])[None, :, None]
    out_ref[...] = dinv * xw_ref[...]


def _tc1b(xw1, degp):
    return pl.pallas_call(
        _k1b_body,
        grid=(NB,),
        in_specs=[
            pl.BlockSpec((2, BN, 128), lambda i: (0, i, 0)),
            pl.BlockSpec((NC, BN, DW), lambda i: (0, i, 0)),
        ],
        out_specs=pl.BlockSpec((2, BN, 128), lambda i: (0, i, 0)),
        out_shape=jax.ShapeDtypeStruct((2, NPAD, 128), jnp.float32),
    )(xw1, degp)


def _k2_body(s1_ref, degp_ref, b1_ref, w2_ref, out_ref):
    dinv = _dinv_of(degp_ref[...])[:, None]
    h0 = jax.nn.relu(dinv * s1_ref[0] + b1_ref[0][None, :])
    h1 = jax.nn.relu(dinv * s1_ref[1] + b1_ref[1][None, :])
    xw = (jnp.dot(h0, w2_ref[0], preferred_element_type=jnp.float32)
          + jnp.dot(h1, w2_ref[1], preferred_element_type=jnp.float32))
    out_ref[...] = dinv * xw


def _tc2(S1, degp, b1r, W2r):
    return pl.pallas_call(
        _k2_body,
        grid=(NB,),
        in_specs=[
            pl.BlockSpec((2, BN, 128), lambda i: (0, i, 0)),
            pl.BlockSpec((NC, BN, DW), lambda i: (0, i, 0)),
            pl.BlockSpec((2, 128), lambda i: (0, 0)),
            pl.BlockSpec((2, 128, 128), lambda i: (0, 0, 0)),
        ],
        out_specs=pl.BlockSpec((BN, 128), lambda i: (i, 0)),
        out_shape=jax.ShapeDtypeStruct((NPAD, 128), jnp.float32),
    )(S1, degp, b1r, W2r)


def _k3_body(p_ref, xs2_ref, degp_ref, b2_ref, out_ref):
    dinv = _dinv_of(degp_ref[...])[:, None]
    agg = p_ref[0] + p_ref[1] - xs2_ref[...]
    out_ref[...] = dinv * agg + b2_ref[0][None, :]


def _tc3(P, xs2, degp, b2r):
    return pl.pallas_call(
        _k3_body,
        grid=(NB,),
        in_specs=[
            pl.BlockSpec((2, BN, 128), lambda i: (0, i, 0)),
            pl.BlockSpec((BN, 128), lambda i: (i, 0)),
            pl.BlockSpec((NC, BN, DW), lambda i: (0, i, 0)),
            pl.BlockSpec((1, 128), lambda i: (0, 0)),
        ],
        out_specs=pl.BlockSpec((BN, 128), lambda i: (i, 0)),
        out_shape=jax.ShapeDtypeStruct((NPAD, 128), jnp.float32),
    )(P, xs2, degp, b2r)


def kernel(x, edge_index, W1, b1, W2, b2):
    row = edge_index[0].astype(jnp.int32)
    col = edge_index[1].astype(jnp.int32)

    rowA = _pad_edges(row, NC * NS, NBLK2).reshape(NC, NS, NBLK2, B)
    colA = _pad_edges(col, NC * NS, NBLK2).reshape(NC, NS, NBLK2, B)
    row1_base = _pad_edges(row, NS, NBLK1)
    row1 = jnp.stack([row1_base, row1_base + NPAD])
    col1_base = _pad_edges(col, NS, NBLK1)
    col1 = jnp.stack([col1_base, col1_base])

    x_pad = jnp.pad(x, ((0, NPAD - N), (0, 0)))
    b1r = b1.reshape(2, 128)
    b2r = b2.reshape(1, 128)
    W2r = W2.reshape(2, 128, 128)

    degp = _deg_kernel(colA, jnp.zeros((NPAD, DW), jnp.float32),
                   jnp.ones((B, DW), jnp.float32))
    xw1 = _tc1a(x_pad, W1)
    xs1 = _tc1b(xw1, degp)
    S1 = _agg1(xs1.reshape(2 * NPAD, 128), row1, col1)
    xs2 = _tc2(S1.reshape(2, NPAD, 128), degp, b1r, W2r)
    P = _agg2(xs2, rowA, colA)
    out = _tc3(P.reshape(2, NPAD, 128), xs2, degp, b2r)
    return out[:N]


# speedup vs baseline: 2.2901x; 1.1139x over previous
import functools

import jax
import jax.numpy as jnp
from jax import lax
from jax.experimental import pallas as pl
from jax.experimental.pallas import tpu as pltpu
from jax.experimental.pallas import tpu_sc as plsc

N = 10000
E = 320000
IN_CH = 128
OUT_CH = 128
HID = 2 * OUT_CH

NC = 2
NS = 16
B = 128

NPAD = 10240
TRASH = N
STRIPE = NPAD // NS

CH = 16
NBLK1 = 160
NPROC1 = 157
NBLK2 = 80
NPROC2 = 79
DW = 128

_mesh = plsc.VectorSubcoreMesh(
    core_axis_name="c", subcore_axis_name="s", num_cores=NC, num_subcores=NS)


def _pad_edges(idx, n_chunks, nblk):
    per = E // n_chunks
    idx = idx.reshape(n_chunks, per)
    idx = jnp.pad(idx, ((0, 0), (0, nblk * B - per)), constant_values=TRASH)
    return idx.reshape(n_chunks, nblk, B)


def _deg_body(col_hbm, zeros_hbm, ones_hbm, degp_hbm, col_v, ones_v, deg_sp,
              sem):
    c = lax.axis_index("c")
    s = lax.axis_index("s")
    base = s * STRIPE
    pltpu.sync_copy(zeros_hbm.at[pl.ds(base, STRIPE)],
                    deg_sp.at[pl.ds(base, STRIPE)])
    pltpu.sync_copy(ones_hbm, ones_v)
    plsc.subcore_barrier()

    nchk = -(-NPROC2 // CH)
    for k in range(nchk):
        pltpu.sync_copy(col_hbm.at[c, s, pl.ds(k * CH, CH)], col_v)
        nb = min(CH, NPROC2 - k * CH)
        descs = [
            pltpu.async_copy(ones_v, deg_sp.at[col_v.at[j]], sem, add=True)
            for j in range(nb)
        ]
        for d in descs:
            d.wait()
    plsc.subcore_barrier()
    pltpu.sync_copy(deg_sp.at[pl.ds(base, STRIPE)],
                    degp_hbm.at[c, pl.ds(base, STRIPE)])


_deg_kernel = functools.partial(
    pl.kernel,
    out_type=jax.ShapeDtypeStruct((NC, NPAD, DW), jnp.float32),
    mesh=_mesh,
    scratch_types=[
        pltpu.VMEM((CH, B), jnp.int32),
        pltpu.VMEM((B, DW), jnp.float32),
        pltpu.VMEM_SHARED((NPAD, DW), jnp.float32),
        pltpu.SemaphoreType.DMA,
    ],
)(_deg_body)


def _agg_body(nblk, init_mul, table_hbm, row_hbm, col_hbm, out_hbm,
              row_v, col_v, buf0, buf1, out_sp, sem0, sem1, isem):
    c = lax.axis_index("c")
    s = lax.axis_index("s")
    base = s * STRIPE
    pltpu.sync_copy(table_hbm.at[pl.ds(init_mul * c * NPAD + base, STRIPE)],
                    out_sp.at[pl.ds(base, STRIPE)])
    pltpu.sync_copy(row_hbm.at[c, s, pl.ds(0, CH)], row_v.at[0])
    pltpu.sync_copy(col_hbm.at[c, s, pl.ds(0, CH)], col_v.at[0])
    plsc.subcore_barrier()
    bufs = (buf0, buf1)
    sems = (sem0, sem1)
    nchk = -(-nblk // CH)
    stage = []
    gat = pltpu.async_copy(table_hbm.at[row_v.at[0, 0]], bufs[0], sems[0])
    for j in range(nblk):
        ch, pos = divmod(j, CH)
        if pos == 0 and ch + 1 < nchk:
            nxt_p = (ch + 1) % 2
            stage = [
                pltpu.async_copy(row_hbm.at[c, s, pl.ds((ch + 1) * CH, CH)],
                                 row_v.at[nxt_p], isem),
                pltpu.async_copy(col_hbm.at[c, s, pl.ds((ch + 1) * CH, CH)],
                                 col_v.at[nxt_p], isem),
            ]
        nxt = None
        if j + 1 < nblk:
            ch1, pos1 = divmod(j + 1, CH)
            if pos1 == 0:
                for d in stage:
                    d.wait()
                stage = []
            nxt = pltpu.async_copy(
                table_hbm.at[row_v.at[ch1 % 2, pos1]], bufs[(j + 1) % 2],
                sems[(j + 1) % 2])
        gat.wait()
        pltpu.sync_copy(bufs[j % 2], out_sp.at[col_v.at[ch % 2, pos]],
                        add=True)
        gat = nxt
    plsc.subcore_barrier()
    pltpu.sync_copy(out_sp.at[pl.ds(base, STRIPE)],
                    out_hbm.at[pl.ds(c * NPAD + base, STRIPE)])


def _make_agg(nblk, init_mul):
    return functools.partial(
        pl.kernel,
        out_type=jax.ShapeDtypeStruct((2 * NPAD, 128), jnp.float32),
        mesh=_mesh,
        scratch_types=[
            pltpu.VMEM((2, CH, B), jnp.int32),
            pltpu.VMEM((2, CH, B), jnp.int32),
            pltpu.VMEM((B, 128), jnp.float32),
            pltpu.VMEM((B, 128), jnp.float32),
            pltpu.VMEM_SHARED((NPAD, 128), jnp.float32),
            pltpu.SemaphoreType.DMA,
            pltpu.SemaphoreType.DMA,
            pltpu.SemaphoreType.DMA,
        ],
    )(functools.partial(_agg_body, nblk, init_mul))


_agg1 = _make_agg(NPROC1, 1)
_agg2 = _make_agg(NPROC2, 1)


BN = 1024
NB = NPAD // BN


def _dinv_of(degp):
    deg = degp[0, :, 0] + degp[1, :, 0] + 1.0
    return lax.rsqrt(deg)


def _k1a_body(x_ref, w1_ref, out_ref):
    out_ref[0] = jnp.dot(x_ref[...], w1_ref[...],
                         preferred_element_type=jnp.float32)


def _tc1a(x_pad, W1):
    return pl.pallas_call(
        _k1a_body,
        grid=(NB, 2),
        in_specs=[
            pl.BlockSpec((BN, IN_CH), lambda i, c: (i, 0)),
            pl.BlockSpec((IN_CH, 128), lambda i, c: (0, c)),
        ],
        out_specs=pl.BlockSpec((1, BN, 128), lambda i, c: (c, i, 0)),
        out_shape=jax.ShapeDtypeStruct((2, NPAD, 128), jnp.float32),
    )(x_pad, W1)


def _k1b_body(xw_ref, degp_ref, out_ref):
    dinv = _dinv_of(degp_ref[...])[None, :, None]
    out_ref[...] = dinv * xw_ref[...]


def _tc1b(xw1, degp):
    return pl.pallas_call(
        _k1b_body,
        grid=(NB,),
        in_specs=[
            pl.BlockSpec((2, BN, 128), lambda i: (0, i, 0)),
            pl.BlockSpec((NC, BN, DW), lambda i: (0, i, 0)),
        ],
        out_specs=pl.BlockSpec((2, BN, 128), lambda i: (0, i, 0)),
        out_shape=jax.ShapeDtypeStruct((2, NPAD, 128), jnp.float32),
    )(xw1, degp)


def _k2_body(s1_ref, degp_ref, b1_ref, w2_ref, out_ref):
    dinv = _dinv_of(degp_ref[...])[:, None]
    h0 = jax.nn.relu(dinv * s1_ref[0] + b1_ref[0][None, :])
    h1 = jax.nn.relu(dinv * s1_ref[1] + b1_ref[1][None, :])
    xw = (jnp.dot(h0, w2_ref[0], preferred_element_type=jnp.float32)
          + jnp.dot(h1, w2_ref[1], preferred_element_type=jnp.float32))
    xs2 = dinv * xw
    out_ref[0] = xs2
    out_ref[1] = xs2


def _tc2(S1, degp, b1r, W2r):
    return pl.pallas_call(
        _k2_body,
        grid=(NB,),
        in_specs=[
            pl.BlockSpec((2, BN, 128), lambda i: (0, i, 0)),
            pl.BlockSpec((NC, BN, DW), lambda i: (0, i, 0)),
            pl.BlockSpec((2, 128), lambda i: (0, 0)),
            pl.BlockSpec((2, 128, 128), lambda i: (0, 0, 0)),
        ],
        out_specs=pl.BlockSpec((2, BN, 128), lambda i: (0, i, 0)),
        out_shape=jax.ShapeDtypeStruct((2, NPAD, 128), jnp.float32),
    )(S1, degp, b1r, W2r)


def _k3_body(p_ref, xs2_ref, degp_ref, b2_ref, out_ref):
    dinv = _dinv_of(degp_ref[...])[:, None]
    agg = p_ref[0] + p_ref[1] - xs2_ref[...]
    out_ref[...] = dinv * agg + b2_ref[0][None, :]


def _tc3(P, xs2, degp, b2r):
    return pl.pallas_call(
        _k3_body,
        grid=(NB,),
        in_specs=[
            pl.BlockSpec((2, BN, 128), lambda i: (0, i, 0)),
            pl.BlockSpec((BN, 128), lambda i: (i, 0)),
            pl.BlockSpec((NC, BN, DW), lambda i: (0, i, 0)),
            pl.BlockSpec((1, 128), lambda i: (0, 0)),
        ],
        out_specs=pl.BlockSpec((BN, 128), lambda i: (i, 0)),
        out_shape=jax.ShapeDtypeStruct((NPAD, 128), jnp.float32),
    )(P, xs2, degp, b2r)


def kernel(x, edge_index, W1, b1, W2, b2):
    row = edge_index[0].astype(jnp.int32)
    col = edge_index[1].astype(jnp.int32)

    rowA = _pad_edges(row, NC * NS, NBLK2).reshape(NC, NS, NBLK2, B)
    colA = _pad_edges(col, NC * NS, NBLK2).reshape(NC, NS, NBLK2, B)
    row1_base = _pad_edges(row, NS, NBLK1)
    row1 = jnp.stack([row1_base, row1_base + NPAD])
    col1_base = _pad_edges(col, NS, NBLK1)
    col1 = jnp.stack([col1_base, col1_base])

    x_pad = jnp.pad(x, ((0, NPAD - N), (0, 0)))
    b1r = b1.reshape(2, 128)
    b2r = b2.reshape(1, 128)
    W2r = W2.reshape(2, 128, 128)

    degp = _deg_kernel(colA, jnp.zeros((NPAD, DW), jnp.float32),
                   jnp.ones((B, DW), jnp.float32))
    xw1 = _tc1a(x_pad, W1)
    xs1 = _tc1b(xw1, degp)
    S1 = _agg1(xs1.reshape(2 * NPAD, 128), row1, col1)
    xs2 = _tc2(S1.reshape(2, NPAD, 128), degp, b1r, W2r)
    rowA2 = rowA + jnp.arange(NC, dtype=jnp.int32).reshape(NC, 1, 1, 1) * NPAD
    P = _agg2(xs2.reshape(2 * NPAD, 128), rowA2, colA)
    out = _tc3(P.reshape(2, NPAD, 128), xs2[0], degp, b2r)
    return out[:N]
